# Initial kernel scaffold; baseline (speedup 1.0000x reference)
import functools

import jax
import jax.numpy as jnp
from jax import lax
from jax.experimental import pallas as pl
from jax.experimental.pallas import tpu as pltpu
from jax.experimental.pallas import tpu_sc as plsc

N = 50000
NPAD = 50176
E = 800000
EPAD = 802816
D = 160
EE_D = 48
NB = 512
EB = 2048
G = 128
NCH = 12544
BUF_ROWS = NCH + G
F32 = jnp.float32

_SC_MESH = plsc.VectorSubcoreMesh(core_axis_name="c", subcore_axis_name="s")


def _ln_s(s):
    mu = jnp.mean(s, axis=1, keepdims=True)
    var = jnp.mean((s - mu) * (s - mu), axis=1, keepdims=True)
    return (s - mu) / jnp.sqrt(var + 1e-5)


def _ln_v(vp, nv):
    msq = (vp[0] * vp[0] + vp[1] * vp[1] + vp[2] * vp[2])
    vn = jnp.sqrt(jnp.sum(msq, axis=1, keepdims=True) / nv + 1e-8)
    return [p / vn for p in vp]


def _dot(a, b):
    return jnp.dot(a, b, preferred_element_type=F32)


def _gvp_core(s_lin, vh, ws_v, bs, wv, wg, bg, relu):
    vn = jnp.sqrt(vh[0] * vh[0] + vh[1] * vh[1] + vh[2] * vh[2] + 1e-8)
    so = s_lin + _dot(vn, ws_v) + bs
    if wv is None:
        return (jnp.maximum(so, 0.0) if relu else so), None
    gate = jax.nn.sigmoid(_dot(so, wg) + bg)
    vout = [_dot(h, wv) * gate for h in vh]
    so = jnp.maximum(so, 0.0) if relu else so
    return so, vout


def _pack_node_row(s, vp, ones_col):
    b = s.shape[0]
    mid = jnp.full((b, 1), 1.0, F32) if ones_col else jnp.zeros((b, 1), F32)
    return jnp.concatenate(
        [s, mid, jnp.zeros((b, 11), F32), vp[0], vp[1], vp[2]], axis=1)


def _split_node_row(t):
    s = t[:, 0:100]
    vp = [t[:, 112 + 16 * c:128 + 16 * c] for c in range(3)]
    return s, vp


def _full(shape):
    return pl.BlockSpec(shape, lambda i: (0,) * len(shape))


def _node_embed_body(s_ref, v_ref, wh, ws_s, ws_v, bs, wv, wg, bg, out_ref):
    s = s_ref[:, 0:6]
    vp = [v_ref[:, 3 * c:3 * c + 3] for c in range(3)]
    s = _ln_s(s)
    vp = _ln_v(vp, 3)
    vh = [_dot(p, wh[...]) for p in vp]
    so, vo = _gvp_core(_dot(s, ws_s[...]), vh, ws_v[...], bs[...],
                       wv[...], wg[...], bg[...], relu=False)
    out_ref[...] = _pack_node_row(so, vo, ones_col=False)


def _edge_embed_body(es_ref, ev_ref, wh, ws_s, ws_v, bs, wv, wg_t, bg, out_ref):
    es = es_ref[...]
    evp = [ev_ref[:, c:c + 1] for c in range(3)]
    es = _ln_s(es)
    evp = _ln_v(evp, 1)
    whs = wh[0, 0]
    vh = [p * whs for p in evp]
    vn = jnp.sqrt(vh[0] * vh[0] + vh[1] * vh[1] + vh[2] * vh[2] + 1e-8)
    so = _dot(es, ws_s[...]) + vn * ws_v[...] + bs[...]
    gate = jax.nn.sigmoid(
        jnp.sum(so * wg_t[...], axis=1, keepdims=True) + bg[0, 0])
    vo = [p * wv[0, 0] * gate for p in vh]
    b = so.shape[0]
    out_ref[...] = jnp.concatenate(
        [so, vo[0], vo[1], vo[2], jnp.zeros((b, 13), F32)], axis=1)


def _edge_msg_body(gs_ref, gd_ref, ee_ref,
                   m0_wh_s, m0_wh_e, m0_wh_d, m0_ws_ss, m0_ws_es, m0_ws_ds,
                   m0_ws_v, m0_bs, m0_wv, m0_wg, m0_bg,
                   m1_wh, m1_ws_s, m1_ws_v, m1_bs, m1_wv, m1_wg, m1_bg,
                   m2_wh, m2_ws_s, m2_ws_v, m2_bs, m2_wv, m2_wg, m2_bg,
                   out_ref):
    ssrc, vs = _split_node_row(gs_ref[...])
    sdst, vd = _split_node_row(gd_ref[...])
    ee = ee_ref[...]
    es = ee[:, 0:32]
    evc = [ee[:, 32 + c:33 + c] for c in range(3)]
    vh = [_dot(vs[c], m0_wh_s[...]) + evc[c] * m0_wh_e[...]
          + _dot(vd[c], m0_wh_d[...]) for c in range(3)]
    s_lin = _dot(ssrc, m0_ws_ss[...]) + _dot(es, m0_ws_es[...]) \
        + _dot(sdst, m0_ws_ds[...])
    s, v = _gvp_core(s_lin, vh, m0_ws_v[...], m0_bs[...],
                     m0_wv[...], m0_wg[...], m0_bg[...], relu=True)
    vh = [_dot(v[c], m1_wh[...]) for c in range(3)]
    s, v = _gvp_core(_dot(s, m1_ws_s[...]), vh, m1_ws_v[...], m1_bs[...],
                     m1_wv[...], m1_wg[...], m1_bg[...], relu=True)
    vh = [_dot(v[c], m2_wh[...]) for c in range(3)]
    s, v = _gvp_core(_dot(s, m2_ws_s[...]), vh, m2_ws_v[...], m2_bs[...],
                     m2_wv[...], m2_wg[...], m2_bg[...], relu=False)
    out_ref[...] = _pack_node_row(s, v, ones_col=True)


def _node_update_body(t_ref, a_ref,
                      f0_wh, f0_ws_s, f0_ws_v, f0_bs, f0_wv, f0_wg, f0_bg,
                      f1_wh, f1_ws_s, f1_ws_v, f1_bs, f1_wv, f1_wg, f1_bg,
                      out_ref):
    s0, vp0 = _split_node_row(t_ref[...])
    agg = a_ref[...]
    deg = jnp.maximum(agg[:, 100:101], 1.0)
    s = s0 + agg[:, 0:100] / deg
    vp = [vp0[c] + agg[:, 112 + 16 * c:128 + 16 * c] / deg for c in range(3)]
    s = _ln_s(s)
    vp = _ln_v(vp, 16)
    vh = [_dot(p, f0_wh[...]) for p in vp]
    fs, fv = _gvp_core(_dot(s, f0_ws_s[...]), vh, f0_ws_v[...], f0_bs[...],
                       f0_wv[...], f0_wg[...], f0_bg[...], relu=True)
    vh = [_dot(fv[c], f1_wh[...]) for c in range(3)]
    fs, fv = _gvp_core(_dot(fs, f1_ws_s[...]), vh, f1_ws_v[...], f1_bs[...],
                       f1_wv[...], f1_wg[...], f1_bg[...], relu=False)
    s = _ln_s(s + fs)
    vp = _ln_v([vp[c] + fv[c] for c in range(3)], 16)
    out_ref[...] = _pack_node_row(s, vp, ones_col=False)


def _out_pool_body(t_ref, b_ref, wh, ws_s, ws_v, bs,
                   res_ref, emb_ref, acc, cnt):
    i = pl.program_id(0)

    @pl.when(i == 0)
    def _init():
        acc[...] = jnp.zeros_like(acc)
        cnt[...] = jnp.zeros_like(cnt)

    s, vp = _split_node_row(t_ref[...])
    s = _ln_s(s)
    vp = _ln_v(vp, 16)
    vh = [_dot(p, wh[...]) for p in vp]
    res, _ = _gvp_core(_dot(s, ws_s[...]), vh, ws_v[...], bs[...],
                       None, None, None, relu=True)
    res_ref[...] = res
    bid = b_ref[0]
    gid = lax.broadcasted_iota(jnp.int32, (64, 1), 0)
    onehot_t = (gid == bid).astype(F32)
    acc[...] = acc[...] + _dot(onehot_t, res)
    cnt[...] = cnt[...] + _dot(onehot_t, jnp.ones((res.shape[0], 8), F32))
    emb_ref[...] = acc[...] / jnp.maximum(cnt[:, 0:1], 1.0)


def _node_embed(ns, nv, w):
    grid = NPAD // NB
    return pl.pallas_call(
        _node_embed_body,
        grid=(grid,),
        in_specs=[pl.BlockSpec((NB, 8), lambda i: (i, 0)),
                  pl.BlockSpec((NB, 16), lambda i: (i, 0))]
        + [_full(x.shape) for x in w],
        out_specs=pl.BlockSpec((NB, D), lambda i: (i, 0)),
        out_shape=jax.ShapeDtypeStruct((NPAD, D), F32),
    )(ns, nv, *w)


def _edge_embed(es, ev, w):
    grid = EPAD // EB
    return pl.pallas_call(
        _edge_embed_body,
        grid=(grid,),
        in_specs=[pl.BlockSpec((EB, 32), lambda i: (i, 0)),
                  pl.BlockSpec((EB, 8), lambda i: (i, 0))]
        + [_full(x.shape) for x in w],
        out_specs=pl.BlockSpec((EB, EE_D), lambda i: (i, 0)),
        out_shape=jax.ShapeDtypeStruct((EPAD, EE_D), F32),
    )(es, ev, *w)


def _edge_msg(gs, gd, ee, w):
    grid = EPAD // EB
    return pl.pallas_call(
        _edge_msg_body,
        grid=(grid,),
        in_specs=[pl.BlockSpec((EB, D), lambda i: (i, 0)),
                  pl.BlockSpec((EB, D), lambda i: (i, 0)),
                  pl.BlockSpec((EB, EE_D), lambda i: (i, 0))]
        + [_full(x.shape) for x in w],
        out_specs=pl.BlockSpec((EB, D), lambda i: (i, 0)),
        out_shape=jax.ShapeDtypeStruct((EPAD, D), F32),
    )(gs, gd, ee, *w)


def _node_update(t, a, w):
    grid = NPAD // NB
    return pl.pallas_call(
        _node_update_body,
        grid=(grid,),
        in_specs=[pl.BlockSpec((NB, D), lambda i: (i, 0)),
                  pl.BlockSpec((NB, D), lambda i: (i, 0))]
        + [_full(x.shape) for x in w],
        out_specs=pl.BlockSpec((NB, D), lambda i: (i, 0)),
        out_shape=jax.ShapeDtypeStruct((NPAD, D), F32),
    )(t, a, *w)


def _out_pool(t, b3, w):
    grid = NPAD // NB
    return pl.pallas_call(
        _out_pool_body,
        grid=(grid,),
        in_specs=[pl.BlockSpec((NB, D), lambda i: (i, 0)),
                  pl.BlockSpec((1, 1, NB), lambda i: (i, 0, 0))]
        + [_full(x.shape) for x in w],
        out_specs=[pl.BlockSpec((NB, 100), lambda i: (i, 0)),
                   pl.BlockSpec((64, 100), lambda i: (0, 0))],
        out_shape=[jax.ShapeDtypeStruct((NPAD, 100), F32),
                   jax.ShapeDtypeStruct((64, 100), F32)],
        scratch_shapes=[pltpu.VMEM((64, 100), F32), pltpu.VMEM((64, 8), F32)],
        compiler_params=pltpu.CompilerParams(
            dimension_semantics=("arbitrary",)),
    )(t, b3, *w)


def _sc_gather(table, idx):
    nrows, dd = table.shape
    per_w = idx.shape[0] // 32

    @functools.partial(
        pl.kernel,
        out_type=jax.ShapeDtypeStruct((idx.shape[0], dd), F32),
        mesh=_SC_MESH,
        scratch_types=[pltpu.VMEM((G,), jnp.int32),
                       pltpu.VMEM((G, dd), F32),
                       pltpu.SemaphoreType.DMA],
    )
    def k(table_hbm, idx_hbm, out_hbm, idx_v, rows_v, sem):
        wid = lax.axis_index("s") * 2 + lax.axis_index("c")
        base = wid * per_w

        def body(i, carry):
            off = base + i * G
            pltpu.sync_copy(idx_hbm.at[pl.ds(off, G)], idx_v)
            pltpu.async_copy(table_hbm.at[idx_v], rows_v, sem).wait()
            pltpu.sync_copy(rows_v, out_hbm.at[pl.ds(off, G)])
            return carry

        lax.fori_loop(0, per_w // G, body, 0)

    return k(table, idx)


def _sc_scatter(msg, dstp, z128):
    per_s = EPAD // 16

    @functools.partial(
        pl.kernel,
        out_type=jax.ShapeDtypeStruct((NPAD, D), F32),
        mesh=_SC_MESH,
        scratch_types=[pltpu.VMEM((G,), jnp.int32),
                       pltpu.VMEM((G,), jnp.int32),
                       pltpu.VMEM((G, D), F32),
                       pltpu.VMEM((G, D), F32),
                       pltpu.VMEM_SHARED((BUF_ROWS, D), F32)],
    )
    def k(msg_hbm, dst_hbm, z_hbm, agg_hbm, ii_v, li_v, rows_v, zbuf, shared):
        c = lax.axis_index("c")
        s = lax.axis_index("s")
        pltpu.sync_copy(z_hbm, zbuf)
        for kk in range(2):
            nb = (c * 2 + kk) * NCH
            for jj in range(7):
                j = s + 16 * jj

                @pl.when(j < BUF_ROWS // G)
                def _zero():
                    pltpu.sync_copy(zbuf, shared.at[pl.ds(j * G, G)])

            plsc.subcore_barrier()

            def body(i, carry):
                off = s * per_s + i * G
                pltpu.sync_copy(dst_hbm.at[pl.ds(off, G)], ii_v)
                pltpu.sync_copy(msg_hbm.at[pl.ds(off, G)], rows_v)
                for j in range(G // 16):
                    x = ii_v[pl.ds(j * 16, 16)]
                    li = x - nb
                    inr = (li >= 0) & (li < NCH)
                    trash = NCH + j * 16 + lax.broadcasted_iota(
                        jnp.int32, (16,), 0)
                    li_v[pl.ds(j * 16, 16)] = jnp.where(inr, li, trash)
                pltpu.sync_copy(rows_v, shared.at[li_v], add=True)
                return carry

            lax.fori_loop(0, per_s // G, body, 0)
            plsc.subcore_barrier()
            for jj in range(7):
                j = s + 16 * jj

                @pl.when(j < NCH // G)
                def _wb():
                    pltpu.sync_copy(shared.at[pl.ds(j * G, G)], rows_v)
                    pltpu.sync_copy(rows_v, agg_hbm.at[pl.ds(nb + j * G, G)])

            plsc.subcore_barrier()

    return k(msg, dstp, z128)


def _prep_weights(params):
    def row(x):
        return x.reshape(1, -1)

    ne = params['node_embed']
    w_ne = (ne['wh'], ne['ws'][0:6], ne['ws'][6:22], row(ne['bs']),
            ne['wv'], ne['wg'], row(ne['bg']))
    ee = params['edge_embed']
    w_ee = (ee['wh'], ee['ws'][0:32], row(ee['ws'][32]), row(ee['bs']),
            ee['wv'], ee['wg'].T, row(ee['bg']))
    w_msg, w_ff = [], []
    for layer in params['layers']:
        m0, m1, m2 = layer['msg']
        wm = (m0['wh'][0:16], m0['wh'][16:17], m0['wh'][17:33],
              m0['ws'][0:100], m0['ws'][100:132], m0['ws'][132:232],
              m0['ws'][232:265], row(m0['bs']), m0['wv'], m0['wg'],
              row(m0['bg']))
        for m in (m1, m2):
            wm = wm + (m['wh'], m['ws'][0:100], m['ws'][100:116],
                       row(m['bs']), m['wv'], m['wg'], row(m['bg']))
        w_msg.append(wm)
        f0, f1 = layer['ff']
        wf = (f0['wh'], f0['ws'][0:100], f0['ws'][100:132], row(f0['bs']),
              f0['wv'], f0['wg'], row(f0['bg']),
              f1['wh'], f1['ws'][0:400], f1['ws'][400:432], row(f1['bs']),
              f1['wv'], f1['wg'], row(f1['bg']))
        w_ff.append(wf)
    po = params['out']
    w_out = (po['wh'], po['ws'][0:100], po['ws'][100:116], row(po['bs']))
    return w_ne, w_ee, w_msg, w_ff, w_out


def kernel(node_s, node_v, edge_s, edge_v, edge_index, batch, params):
    i32 = jnp.int32
    w_ne, w_ee, w_msg, w_ff, w_out = _prep_weights(params)

    ns = jnp.pad(node_s, ((0, NPAD - N), (0, 2)))
    nv = jnp.pad(node_v.transpose(0, 2, 1).reshape(N, 9),
                 ((0, NPAD - N), (0, 7)))
    es = jnp.pad(edge_s, ((0, EPAD - E), (0, 0)))
    ev = jnp.pad(edge_v.reshape(E, 3), ((0, EPAD - E), (0, 5)))

    pad_ids = jnp.arange(EPAD - E, dtype=i32) % N
    srcp = jnp.concatenate([edge_index[0], pad_ids])
    dst_g = jnp.concatenate([edge_index[1], pad_ids])
    dst_s = jnp.concatenate([edge_index[1],
                             jnp.full((EPAD - E,), -1, i32)])
    b3 = jnp.pad(batch, (0, NPAD - N),
                 constant_values=-1).reshape(NPAD // NB, 1, NB)
    z128 = jnp.zeros((G, D), F32)

    t = _node_embed(ns, nv, w_ne)
    eemb = _edge_embed(es, ev, w_ee)
    for l in range(3):
        g_src = _sc_gather(t, srcp)
        g_dst = _sc_gather(t, dst_g)
        m = _edge_msg(g_src, g_dst, eemb, w_msg[l])
        agg = _sc_scatter(m, dst_s, z128)
        t = _node_update(t, agg, w_ff[l])
    residue_pad, graph_emb = _out_pool(t, b3, w_out)
    return (graph_emb, residue_pad[:N])


# speedup vs baseline: 11.7404x; 11.7404x over previous
import functools

import jax
import jax.numpy as jnp
from jax import lax
from jax.experimental import pallas as pl
from jax.experimental.pallas import tpu as pltpu
from jax.experimental.pallas import tpu_sc as plsc

N = 50000
NPAD = 50176
E = 800000
EPAD = 802816
D = 256
EE_D = 128
NB = 512
EB = 2048
G = 128
SB = 256
NSB = EPAD // SB
NR = 75264
ZR = 53504
F32 = jnp.float32

@functools.cache
def _sc_mesh():
    return plsc.VectorSubcoreMesh(core_axis_name="c", subcore_axis_name="s")


def _ln_s(s):
    mu = jnp.mean(s, axis=1, keepdims=True)
    var = jnp.mean((s - mu) * (s - mu), axis=1, keepdims=True)
    return (s - mu) / jnp.sqrt(var + 1e-5)


def _ln_v(vp, nv):
    msq = (vp[0] * vp[0] + vp[1] * vp[1] + vp[2] * vp[2])
    vn = jnp.sqrt(jnp.sum(msq, axis=1, keepdims=True) / nv + 1e-8)
    return [p / vn for p in vp]


def _dot(a, b):
    return jnp.dot(a, b, preferred_element_type=F32)


def _gvp_core(s_lin, vh, ws_v, bs, wv, wg, bg, relu):
    vn = jnp.sqrt(vh[0] * vh[0] + vh[1] * vh[1] + vh[2] * vh[2] + 1e-8)
    so = s_lin + _dot(vn, ws_v) + bs
    if wv is None:
        return (jnp.maximum(so, 0.0) if relu else so), None
    gate = jax.nn.sigmoid(_dot(so, wg) + bg)
    vout = [_dot(h, wv) * gate for h in vh]
    so = jnp.maximum(so, 0.0) if relu else so
    return so, vout


def _pack_node_row(s, vp, ones_col):
    b = s.shape[0]
    mid = jnp.full((b, 1), 1.0, F32) if ones_col else jnp.zeros((b, 1), F32)
    return jnp.concatenate(
        [s, mid, jnp.zeros((b, 11), F32), vp[0], vp[1], vp[2],
         jnp.zeros((b, D - 160), F32)], axis=1)


def _split_node_row(t):
    s = t[:, 0:100]
    vp = [t[:, 112 + 16 * c:128 + 16 * c] for c in range(3)]
    return s, vp


def _full(shape):
    return pl.BlockSpec(shape, lambda i: (0,) * len(shape))


def _node_embed_body(s_ref, v_ref, wh, ws_s, ws_v, bs, wv, wg, bg, out_ref):
    s = s_ref[:, 0:6]
    vp = [v_ref[:, 3 * c:3 * c + 3] for c in range(3)]
    s = _ln_s(s)
    vp = _ln_v(vp, 3)
    vh = [_dot(p, wh[...]) for p in vp]
    so, vo = _gvp_core(_dot(s, ws_s[...]), vh, ws_v[...], bs[...],
                       wv[...], wg[...], bg[...], relu=False)
    out_ref[...] = _pack_node_row(so, vo, ones_col=False)


def _edge_embed_body(es_ref, ev_ref, wh, ws_s, ws_v, bs, wv, wg_t, bg, out_ref):
    es = es_ref[...]
    evp = [ev_ref[:, c:c + 1] for c in range(3)]
    es = _ln_s(es)
    evp = _ln_v(evp, 1)
    whs = wh[0, 0]
    vh = [p * whs for p in evp]
    vn = jnp.sqrt(vh[0] * vh[0] + vh[1] * vh[1] + vh[2] * vh[2] + 1e-8)
    so = _dot(es, ws_s[...]) + vn * ws_v[...] + bs[...]
    gate = jax.nn.sigmoid(
        jnp.sum(so * wg_t[...], axis=1, keepdims=True) + bg[0, 0])
    vo = [p * wv[0, 0] * gate for p in vh]
    b = so.shape[0]
    out_ref[...] = jnp.concatenate(
        [so, vo[0], vo[1], vo[2], jnp.zeros((b, EE_D - 35), F32)], axis=1)


def _edge_msg_body(gs_ref, gd_ref, ee_ref, rank_ref,
                   m0_wh_s, m0_wh_e, m0_wh_d, m0_ws_ss, m0_ws_es, m0_ws_ds,
                   m0_ws_v, m0_bs, m0_wv, m0_wg, m0_bg,
                   m1_wh, m1_ws_s, m1_ws_v, m1_bs, m1_wv, m1_wg, m1_bg,
                   m2_wh, m2_ws_s, m2_ws_v, m2_bs, m2_wv, m2_wg, m2_bg,
                   out_ref):
    ssrc, vs = _split_node_row(gs_ref[...])
    sdst, vd = _split_node_row(gd_ref[...])
    ee = ee_ref[...]
    es = ee[:, 0:32]
    evc = [ee[:, 32 + c:33 + c] for c in range(3)]
    vh = [_dot(vs[c], m0_wh_s[...]) + evc[c] * m0_wh_e[...]
          + _dot(vd[c], m0_wh_d[...]) for c in range(3)]
    s_lin = _dot(ssrc, m0_ws_ss[...]) + _dot(es, m0_ws_es[...]) \
        + _dot(sdst, m0_ws_ds[...])
    s, v = _gvp_core(s_lin, vh, m0_ws_v[...], m0_bs[...],
                     m0_wv[...], m0_wg[...], m0_bg[...], relu=True)
    vh = [_dot(v[c], m1_wh[...]) for c in range(3)]
    s, v = _gvp_core(_dot(s, m1_ws_s[...]), vh, m1_ws_v[...], m1_bs[...],
                     m1_wv[...], m1_wg[...], m1_bg[...], relu=True)
    vh = [_dot(v[c], m2_wh[...]) for c in range(3)]
    s, v = _gvp_core(_dot(s, m2_ws_s[...]), vh, m2_ws_v[...], m2_bs[...],
                     m2_wv[...], m2_wg[...], m2_bg[...], relu=False)
    msgs = _pack_node_row(s, v, ones_col=True)
    rk = rank_ref[0, 0, :]
    riota = lax.broadcasted_iota(jnp.int32, (SB, 1), 0)
    parts = []
    for s8 in range(EB // SB):
        oh = (riota == rk[s8 * SB:(s8 + 1) * SB].reshape(1, SB)).astype(F32)
        parts.append(_dot(oh, msgs[s8 * SB:(s8 + 1) * SB, :]))
    out_ref[...] = jnp.concatenate(parts, axis=0)


def _node_update_body(t_ref, a0_ref, a1_ref, c0_ref, c1_ref, cid_ref,
                      f0_wh, f0_ws_s, f0_ws_v, f0_bs, f0_wv, f0_wg, f0_bg,
                      f1_wh, f1_ws_s, f1_ws_v, f1_bs, f1_wv, f1_wg, f1_bg,
                      out_ref):
    s0, vp0 = _split_node_row(t_ref[...])
    car = c0_ref[...] + c1_ref[...]
    ids = cid_ref[0]
    base = pl.program_id(0) * NB
    oh = (lax.broadcasted_iota(jnp.int32, (NB, 1), 0) + base == ids)
    agg = a0_ref[...] + a1_ref[...] + _dot(oh.astype(F32), car)
    deg = jnp.maximum(agg[:, 100:101], 1.0)
    s = s0 + agg[:, 0:100] / deg
    vp = [vp0[c] + agg[:, 112 + 16 * c:128 + 16 * c] / deg for c in range(3)]
    s = _ln_s(s)
    vp = _ln_v(vp, 16)
    vh = [_dot(p, f0_wh[...]) for p in vp]
    fs, fv = _gvp_core(_dot(s, f0_ws_s[...]), vh, f0_ws_v[...], f0_bs[...],
                       f0_wv[...], f0_wg[...], f0_bg[...], relu=True)
    vh = [_dot(fv[c], f1_wh[...]) for c in range(3)]
    fs, fv = _gvp_core(_dot(fs, f1_ws_s[...]), vh, f1_ws_v[...], f1_bs[...],
                       f1_wv[...], f1_wg[...], f1_bg[...], relu=False)
    s = _ln_s(s + fs)
    vp = _ln_v([vp[c] + fv[c] for c in range(3)], 16)
    out_ref[...] = _pack_node_row(s, vp, ones_col=False)


def _out_pool_body(t_ref, b_ref, wh, ws_s, ws_v, bs,
                   res_ref, emb_ref, acc, cnt):
    i = pl.program_id(0)

    @pl.when(i == 0)
    def _init():
        acc[...] = jnp.zeros_like(acc)
        cnt[...] = jnp.zeros_like(cnt)

    s, vp = _split_node_row(t_ref[...])
    s = _ln_s(s)
    vp = _ln_v(vp, 16)
    vh = [_dot(p, wh[...]) for p in vp]
    res, _ = _gvp_core(_dot(s, ws_s[...]), vh, ws_v[...], bs[...],
                       None, None, None, relu=True)
    res_ref[...] = res
    bid = b_ref[0]
    gid = lax.broadcasted_iota(jnp.int32, (64, 1), 0)
    onehot_t = (gid == bid).astype(F32)
    acc[...] = acc[...] + _dot(onehot_t, res)
    cnt[...] = cnt[...] + _dot(onehot_t, jnp.ones((res.shape[0], 8), F32))
    emb_ref[...] = acc[...] / jnp.maximum(cnt[:, 0:1], 1.0)


def _node_embed(ns, nv, w):
    grid = NPAD // NB
    return pl.pallas_call(
        _node_embed_body,
        grid=(grid,),
        in_specs=[pl.BlockSpec((NB, 8), lambda i: (i, 0)),
                  pl.BlockSpec((NB, 16), lambda i: (i, 0))]
        + [_full(x.shape) for x in w],
        out_specs=pl.BlockSpec((NB, D), lambda i: (i, 0)),
        out_shape=jax.ShapeDtypeStruct((NPAD, D), F32),
    )(ns, nv, *w)


def _edge_embed(es, ev, w):
    grid = EPAD // EB
    return pl.pallas_call(
        _edge_embed_body,
        grid=(grid,),
        in_specs=[pl.BlockSpec((EB, 32), lambda i: (i, 0)),
                  pl.BlockSpec((EB, 8), lambda i: (i, 0))]
        + [_full(x.shape) for x in w],
        out_specs=pl.BlockSpec((EB, EE_D), lambda i: (i, 0)),
        out_shape=jax.ShapeDtypeStruct((EPAD, EE_D), F32),
    )(es, ev, *w)


def _edge_msg(gs, gd, ee, rank3, w):
    grid = EPAD // EB
    return pl.pallas_call(
        _edge_msg_body,
        grid=(grid,),
        in_specs=[pl.BlockSpec((EB, D), lambda i: (i, 0)),
                  pl.BlockSpec((EB, D), lambda i: (i, 0)),
                  pl.BlockSpec((EB, EE_D), lambda i: (i, 0)),
                  pl.BlockSpec((1, 1, EB), lambda i: (i, 0, 0))]
        + [_full(x.shape) for x in w],
        out_specs=pl.BlockSpec((EB, D), lambda i: (i, 0)),
        out_shape=jax.ShapeDtypeStruct((EPAD, D), F32),
    )(gs, gd, ee, rank3, *w)


def _node_update(t, agg2, cid3, w):
    grid = NPAD // NB
    return pl.pallas_call(
        _node_update_body,
        grid=(grid,),
        in_specs=[pl.BlockSpec((NB, D), lambda i: (i, 0)),
                  pl.BlockSpec((NB, D), lambda i: (i, 0)),
                  pl.BlockSpec((NB, D), lambda i: (i + NR // NB, 0)),
                  pl.BlockSpec((NSB, D), lambda i: (NPAD // NSB, 0)),
                  pl.BlockSpec((NSB, D), lambda i: ((NR + NPAD) // NSB, 0)),
                  pl.BlockSpec((1, 1, NSB), lambda i: (0, 0, 0))]
        + [_full(x.shape) for x in w],
        out_specs=pl.BlockSpec((NB, D), lambda i: (i, 0)),
        out_shape=jax.ShapeDtypeStruct((NPAD, D), F32),
    )(t, agg2, agg2, agg2, agg2, cid3, *w)


def _out_pool(t, b3, w):
    grid = NPAD // NB
    return pl.pallas_call(
        _out_pool_body,
        grid=(grid,),
        in_specs=[pl.BlockSpec((NB, D), lambda i: (i, 0)),
                  pl.BlockSpec((1, 1, NB), lambda i: (i, 0, 0))]
        + [_full(x.shape) for x in w],
        out_specs=[pl.BlockSpec((NB, 100), lambda i: (i, 0)),
                   pl.BlockSpec((64, 100), lambda i: (0, 0))],
        out_shape=[jax.ShapeDtypeStruct((NPAD, 100), F32),
                   jax.ShapeDtypeStruct((64, 100), F32)],
        scratch_shapes=[pltpu.VMEM((64, 100), F32), pltpu.VMEM((64, 8), F32)],
        compiler_params=pltpu.CompilerParams(
            dimension_semantics=("arbitrary",)),
    )(t, b3, *w)


def _sc_gather(table, idx):
    nrows, dd = table.shape
    per_w = idx.shape[0] // 32

    @functools.partial(
        pl.kernel,
        out_type=jax.ShapeDtypeStruct((idx.shape[0], dd), F32),
        mesh=_sc_mesh(),
        scratch_types=[pltpu.VMEM((G,), jnp.int32),
                       pltpu.VMEM((G, dd), F32),
                       pltpu.SemaphoreType.DMA],
    )
    def k(table_hbm, idx_hbm, out_hbm, idx_v, rows_v, sem):
        wid = lax.axis_index("s") * 2 + lax.axis_index("c")
        base = wid * per_w

        def body(i, carry):
            off = base + i * G
            pltpu.sync_copy(idx_hbm.at[pl.ds(off, G)], idx_v)
            pltpu.async_copy(table_hbm.at[idx_v], rows_v, sem).wait()
            pltpu.sync_copy(rows_v, out_hbm.at[pl.ds(off, G)])
            return carry

        lax.fori_loop(0, per_w // G, body, 0)

    return k(table, idx)


def _sc_scatter(parts, tgt, z128):
    per_w = EPAD // 32
    zrows = ZR // 16

    @functools.partial(
        pl.kernel,
        out_type=jax.ShapeDtypeStruct((2 * NR, D), F32),
        mesh=_sc_mesh(),
        scratch_types=[pltpu.VMEM((G,), jnp.int32),
                       pltpu.VMEM((G,), jnp.int32),
                       pltpu.VMEM((G, D), F32),
                       pltpu.VMEM((G, D), F32),
                       pltpu.SemaphoreType.DMA],
    )
    def k(p_hbm, tgt_hbm, z_hbm, agg_hbm, ii_v, li_v, rows_v, zbuf, sem):
        c = lax.axis_index("c")
        s = lax.axis_index("s")
        base = c * NR
        pltpu.sync_copy(z_hbm, zbuf)
        for j in range(zrows // G + 1):
            st = base + s * zrows + min(j * G, zrows - G)
            pltpu.sync_copy(zbuf, agg_hbm.at[pl.ds(st, G)])
        plsc.subcore_barrier()

        def body(i, carry):
            off = (c * 16 + s) * per_w + i * G
            pltpu.sync_copy(tgt_hbm.at[pl.ds(off, G)], ii_v)
            pltpu.sync_copy(p_hbm.at[pl.ds(off, G)], rows_v)
            for j in range(G // 16):
                li_v[pl.ds(j * 16, 16)] = ii_v[pl.ds(j * 16, 16)] + base
            pltpu.async_copy(rows_v, agg_hbm.at[li_v], sem).wait()
            return carry

        lax.fori_loop(0, per_w // G, body, 0)

    return k(parts, tgt, z128)


def _prep_weights(params):
    def row(x):
        return x.reshape(1, -1)

    ne = params['node_embed']
    w_ne = (ne['wh'], ne['ws'][0:6], ne['ws'][6:22], row(ne['bs']),
            ne['wv'], ne['wg'], row(ne['bg']))
    ee = params['edge_embed']
    w_ee = (ee['wh'], ee['ws'][0:32], row(ee['ws'][32]), row(ee['bs']),
            ee['wv'], ee['wg'].T, row(ee['bg']))
    w_msg, w_ff = [], []
    for layer in params['layers']:
        m0, m1, m2 = layer['msg']
        wm = (m0['wh'][0:16], m0['wh'][16:17], m0['wh'][17:33],
              m0['ws'][0:100], m0['ws'][100:132], m0['ws'][132:232],
              m0['ws'][232:265], row(m0['bs']), m0['wv'], m0['wg'],
              row(m0['bg']))
        for m in (m1, m2):
            wm = wm + (m['wh'], m['ws'][0:100], m['ws'][100:116],
                       row(m['bs']), m['wv'], m['wg'], row(m['bg']))
        w_msg.append(wm)
        f0, f1 = layer['ff']
        wf = (f0['wh'], f0['ws'][0:100], f0['ws'][100:132], row(f0['bs']),
              f0['wv'], f0['wg'], row(f0['bg']),
              f1['wh'], f1['ws'][0:400], f1['ws'][400:432], row(f1['bs']),
              f1['wv'], f1['wg'], row(f1['bg']))
        w_ff.append(wf)
    po = params['out']
    w_out = (po['wh'], po['ws'][0:100], po['ws'][100:116], row(po['bs']))
    return w_ne, w_ee, w_msg, w_ff, w_out


def kernel(node_s, node_v, edge_s, edge_v, edge_index, batch, params):
    i32 = jnp.int32
    w_ne, w_ee, w_msg, w_ff, w_out = _prep_weights(params)

    ns = jnp.pad(node_s, ((0, NPAD - N), (0, 2)))
    nv = jnp.pad(node_v.transpose(0, 2, 1).reshape(N, 9),
                 ((0, NPAD - N), (0, 7)))
    es = jnp.pad(edge_s, ((0, EPAD - E), (0, 0)))
    ev = jnp.pad(edge_v.reshape(E, 3), ((0, EPAD - E), (0, 5)))

    pad_ids = jnp.arange(EPAD - E, dtype=i32) % N
    srcp = jnp.concatenate([edge_index[0], pad_ids])
    dst_p = jnp.concatenate([edge_index[1],
                             jnp.full((EPAD - E,), NPAD - 1, i32)])
    dst_sorted, order = lax.sort_key_val(dst_p, jnp.arange(EPAD, dtype=i32))
    src_sorted = jnp.take(srcp, order)
    e_idx = jnp.arange(EPAD, dtype=i32)
    prev = jnp.concatenate([dst_sorted[:1] - 1, dst_sorted[:-1]])
    newseg = dst_sorted != prev
    b = (newseg & (e_idx % SB != 0)).astype(i32)
    csum = jnp.cumsum(b)
    rank = csum - jnp.take(csum, e_idx - (e_idx % SB))
    sbi = e_idx // SB
    nid = jnp.full((NSB, SB), -1, i32).at[sbi, rank].max(dst_sorted)
    continued = ~newseg[::SB]
    riota = jnp.arange(SB, dtype=i32)[None, :]
    tgt2 = jnp.where(nid < 0, NPAD + NSB + (riota % G), nid)
    tgt2 = tgt2.at[:, 0].set(jnp.where(
        continued, NPAD + jnp.arange(NSB, dtype=i32), tgt2[:, 0]))
    tgt = tgt2.reshape(EPAD)
    cid3 = dst_sorted[::SB].reshape(1, 1, NSB)
    rank3 = rank.reshape(EPAD // EB, 1, EB)
    b3 = jnp.pad(batch, (0, NPAD - N),
                 constant_values=-1).reshape(NPAD // NB, 1, NB)
    z128 = jnp.zeros((G, D), F32)

    t = _node_embed(ns, nv, w_ne)
    eemb = _sc_gather(_edge_embed(es, ev, w_ee), order)
    for l in range(3):
        g_src = _sc_gather(t, src_sorted)
        g_dst = _sc_gather(t, dst_sorted)
        m = _edge_msg(g_src, g_dst, eemb, rank3, w_msg[l])
        agg2 = _sc_scatter(m, tgt, z128)
        t = _node_update(t, agg2, cid3, w_ff[l])
    residue_pad, graph_emb = _out_pool(t, b3, w_out)
    return (graph_emb, residue_pad[:N])


# speedup vs baseline: 12.6887x; 1.0808x over previous
import functools

import jax
import jax.numpy as jnp
from jax import lax
from jax.experimental import pallas as pl
from jax.experimental.pallas import tpu as pltpu
from jax.experimental.pallas import tpu_sc as plsc

N = 50000
NPAD = 50176
E = 800000
EPAD = 802816
D = 256
EE_D = 128
NB = 512
EB = 2048
G = 128
SB = 256
NSB = EPAD // SB
NR = 75264
ZR = 53504
F32 = jnp.float32

@functools.cache
def _sc_mesh():
    return plsc.VectorSubcoreMesh(core_axis_name="c", subcore_axis_name="s")


def _ln_s(s):
    mu = jnp.mean(s, axis=1, keepdims=True)
    var = jnp.mean((s - mu) * (s - mu), axis=1, keepdims=True)
    return (s - mu) / jnp.sqrt(var + 1e-5)


def _ln_v(vp, nv):
    msq = (vp[0] * vp[0] + vp[1] * vp[1] + vp[2] * vp[2])
    vn = jnp.sqrt(jnp.sum(msq, axis=1, keepdims=True) / nv + 1e-8)
    return [p / vn for p in vp]


def _dot(a, b):
    return jnp.dot(a, b, preferred_element_type=F32)


def _gvp_core(s_lin, vh, ws_v, bs, wv, wg, bg, relu):
    vn = jnp.sqrt(vh[0] * vh[0] + vh[1] * vh[1] + vh[2] * vh[2] + 1e-8)
    so = s_lin + _dot(vn, ws_v) + bs
    if wv is None:
        return (jnp.maximum(so, 0.0) if relu else so), None
    gate = jax.nn.sigmoid(_dot(so, wg) + bg)
    vout = [_dot(h, wv) * gate for h in vh]
    so = jnp.maximum(so, 0.0) if relu else so
    return so, vout


def _pack_node_row(s, vp, ones_col):
    b = s.shape[0]
    mid = jnp.full((b, 1), 1.0, F32) if ones_col else jnp.zeros((b, 1), F32)
    return jnp.concatenate(
        [s, mid, jnp.zeros((b, 11), F32), vp[0], vp[1], vp[2],
         jnp.zeros((b, D - 160), F32)], axis=1)


def _split_node_row(t):
    s = t[:, 0:100]
    vp = [t[:, 112 + 16 * c:128 + 16 * c] for c in range(3)]
    return s, vp


def _full(shape):
    return pl.BlockSpec(shape, lambda i: (0,) * len(shape))


def _node_embed_body(s_ref, v_ref, wh, ws_s, ws_v, bs, wv, wg, bg, out_ref):
    s = s_ref[:, 0:6]
    vp = [v_ref[:, 3 * c:3 * c + 3] for c in range(3)]
    s = _ln_s(s)
    vp = _ln_v(vp, 3)
    vh = [_dot(p, wh[...]) for p in vp]
    so, vo = _gvp_core(_dot(s, ws_s[...]), vh, ws_v[...], bs[...],
                       wv[...], wg[...], bg[...], relu=False)
    out_ref[...] = _pack_node_row(so, vo, ones_col=False)


def _edge_embed_body(es_ref, ev_ref, wh, ws_s, ws_v, bs, wv, wg_t, bg, out_ref):
    es = es_ref[...]
    evp = [ev_ref[:, c:c + 1] for c in range(3)]
    es = _ln_s(es)
    evp = _ln_v(evp, 1)
    whs = wh[0, 0]
    vh = [p * whs for p in evp]
    vn = jnp.sqrt(vh[0] * vh[0] + vh[1] * vh[1] + vh[2] * vh[2] + 1e-8)
    so = _dot(es, ws_s[...]) + vn * ws_v[...] + bs[...]
    gate = jax.nn.sigmoid(
        jnp.sum(so * wg_t[...], axis=1, keepdims=True) + bg[0, 0])
    vo = [p * wv[0, 0] * gate for p in vh]
    b = so.shape[0]
    out_ref[...] = jnp.concatenate(
        [so, vo[0], vo[1], vo[2], jnp.zeros((b, EE_D - 35), F32)], axis=1)


def _edge_msg_body(gs_ref, gd_ref, ee_ref, rank_ref,
                   m0_wh_s, m0_wh_e, m0_wh_d, m0_ws_ss, m0_ws_es, m0_ws_ds,
                   m0_ws_v, m0_bs, m0_wv, m0_wg, m0_bg,
                   m1_wh, m1_ws_s, m1_ws_v, m1_bs, m1_wv, m1_wg, m1_bg,
                   m2_wh, m2_ws_s, m2_ws_v, m2_bs, m2_wv, m2_wg, m2_bg,
                   out_ref):
    ssrc, vs = _split_node_row(gs_ref[...])
    sdst, vd = _split_node_row(gd_ref[...])
    ee = ee_ref[...]
    es = ee[:, 0:32]
    evc = [ee[:, 32 + c:33 + c] for c in range(3)]
    vh = [_dot(vs[c], m0_wh_s[...]) + evc[c] * m0_wh_e[...]
          + _dot(vd[c], m0_wh_d[...]) for c in range(3)]
    s_lin = _dot(ssrc, m0_ws_ss[...]) + _dot(es, m0_ws_es[...]) \
        + _dot(sdst, m0_ws_ds[...])
    s, v = _gvp_core(s_lin, vh, m0_ws_v[...], m0_bs[...],
                     m0_wv[...], m0_wg[...], m0_bg[...], relu=True)
    vh = [_dot(v[c], m1_wh[...]) for c in range(3)]
    s, v = _gvp_core(_dot(s, m1_ws_s[...]), vh, m1_ws_v[...], m1_bs[...],
                     m1_wv[...], m1_wg[...], m1_bg[...], relu=True)
    vh = [_dot(v[c], m2_wh[...]) for c in range(3)]
    s, v = _gvp_core(_dot(s, m2_ws_s[...]), vh, m2_ws_v[...], m2_bs[...],
                     m2_wv[...], m2_wg[...], m2_bg[...], relu=False)
    msgs = _pack_node_row(s, v, ones_col=True)
    rk = rank_ref[0, 0, :]
    riota = lax.broadcasted_iota(jnp.int32, (SB, 1), 0)
    parts = []
    for s8 in range(EB // SB):
        oh = (riota == rk[s8 * SB:(s8 + 1) * SB].reshape(1, SB)).astype(F32)
        parts.append(_dot(oh, msgs[s8 * SB:(s8 + 1) * SB, :]))
    out_ref[...] = jnp.concatenate(parts, axis=0)


def _node_update_body(t_ref, a0_ref, a1_ref, c0_ref, c1_ref, cid_ref,
                      f0_wh, f0_ws_s, f0_ws_v, f0_bs, f0_wv, f0_wg, f0_bg,
                      f1_wh, f1_ws_s, f1_ws_v, f1_bs, f1_wv, f1_wg, f1_bg,
                      out_ref):
    s0, vp0 = _split_node_row(t_ref[...])
    car = c0_ref[...] + c1_ref[...]
    ids = cid_ref[0]
    base = pl.program_id(0) * NB
    oh = (lax.broadcasted_iota(jnp.int32, (NB, 1), 0) + base == ids)
    agg = a0_ref[...] + a1_ref[...] + _dot(oh.astype(F32), car)
    deg = jnp.maximum(agg[:, 100:101], 1.0)
    s = s0 + agg[:, 0:100] / deg
    vp = [vp0[c] + agg[:, 112 + 16 * c:128 + 16 * c] / deg for c in range(3)]
    s = _ln_s(s)
    vp = _ln_v(vp, 16)
    vh = [_dot(p, f0_wh[...]) for p in vp]
    fs, fv = _gvp_core(_dot(s, f0_ws_s[...]), vh, f0_ws_v[...], f0_bs[...],
                       f0_wv[...], f0_wg[...], f0_bg[...], relu=True)
    vh = [_dot(fv[c], f1_wh[...]) for c in range(3)]
    fs, fv = _gvp_core(_dot(fs, f1_ws_s[...]), vh, f1_ws_v[...], f1_bs[...],
                       f1_wv[...], f1_wg[...], f1_bg[...], relu=False)
    s = _ln_s(s + fs)
    vp = _ln_v([vp[c] + fv[c] for c in range(3)], 16)
    out_ref[...] = _pack_node_row(s, vp, ones_col=False)


def _out_pool_body(t_ref, b_ref, wh, ws_s, ws_v, bs,
                   res_ref, emb_ref, acc, cnt):
    i = pl.program_id(0)

    @pl.when(i == 0)
    def _init():
        acc[...] = jnp.zeros_like(acc)
        cnt[...] = jnp.zeros_like(cnt)

    s, vp = _split_node_row(t_ref[...])
    s = _ln_s(s)
    vp = _ln_v(vp, 16)
    vh = [_dot(p, wh[...]) for p in vp]
    res, _ = _gvp_core(_dot(s, ws_s[...]), vh, ws_v[...], bs[...],
                       None, None, None, relu=True)
    res_ref[...] = res
    bid = b_ref[0]
    gid = lax.broadcasted_iota(jnp.int32, (64, 1), 0)
    onehot_t = (gid == bid).astype(F32)
    acc[...] = acc[...] + _dot(onehot_t, res)
    cnt[...] = cnt[...] + _dot(onehot_t, jnp.ones((res.shape[0], 8), F32))
    emb_ref[...] = acc[...] / jnp.maximum(cnt[:, 0:1], 1.0)


def _node_embed(ns, nv, w):
    grid = NPAD // NB
    return pl.pallas_call(
        _node_embed_body,
        grid=(grid,),
        in_specs=[pl.BlockSpec((NB, 8), lambda i: (i, 0)),
                  pl.BlockSpec((NB, 16), lambda i: (i, 0))]
        + [_full(x.shape) for x in w],
        out_specs=pl.BlockSpec((NB, D), lambda i: (i, 0)),
        out_shape=jax.ShapeDtypeStruct((NPAD, D), F32),
    )(ns, nv, *w)


def _edge_embed(es, ev, w):
    grid = EPAD // EB
    return pl.pallas_call(
        _edge_embed_body,
        grid=(grid,),
        in_specs=[pl.BlockSpec((EB, 32), lambda i: (i, 0)),
                  pl.BlockSpec((EB, 8), lambda i: (i, 0))]
        + [_full(x.shape) for x in w],
        out_specs=pl.BlockSpec((EB, EE_D), lambda i: (i, 0)),
        out_shape=jax.ShapeDtypeStruct((EPAD, EE_D), F32),
    )(es, ev, *w)


def _edge_msg(gs, gd, ee, rank3, w):
    grid = EPAD // EB
    return pl.pallas_call(
        _edge_msg_body,
        grid=(grid,),
        in_specs=[pl.BlockSpec((EB, D), lambda i: (i, 0)),
                  pl.BlockSpec((EB, D), lambda i: (i, 0)),
                  pl.BlockSpec((EB, EE_D), lambda i: (i, 0)),
                  pl.BlockSpec((1, 1, EB), lambda i: (i, 0, 0))]
        + [_full(x.shape) for x in w],
        out_specs=pl.BlockSpec((EB, D), lambda i: (i, 0)),
        out_shape=jax.ShapeDtypeStruct((EPAD, D), F32),
    )(gs, gd, ee, rank3, *w)


def _node_update(t, agg2, cid3, w):
    grid = NPAD // NB
    return pl.pallas_call(
        _node_update_body,
        grid=(grid,),
        in_specs=[pl.BlockSpec((NB, D), lambda i: (i, 0)),
                  pl.BlockSpec((NB, D), lambda i: (i, 0)),
                  pl.BlockSpec((NB, D), lambda i: (i + NR // NB, 0)),
                  pl.BlockSpec((NSB, D), lambda i: (NPAD // NSB, 0)),
                  pl.BlockSpec((NSB, D), lambda i: ((NR + NPAD) // NSB, 0)),
                  pl.BlockSpec((1, 1, NSB), lambda i: (0, 0, 0))]
        + [_full(x.shape) for x in w],
        out_specs=pl.BlockSpec((NB, D), lambda i: (i, 0)),
        out_shape=jax.ShapeDtypeStruct((NPAD, D), F32),
    )(t, agg2, agg2, agg2, agg2, cid3, *w)


def _out_pool(t, b3, w):
    grid = NPAD // NB
    return pl.pallas_call(
        _out_pool_body,
        grid=(grid,),
        in_specs=[pl.BlockSpec((NB, D), lambda i: (i, 0)),
                  pl.BlockSpec((1, 1, NB), lambda i: (i, 0, 0))]
        + [_full(x.shape) for x in w],
        out_specs=[pl.BlockSpec((NB, 100), lambda i: (i, 0)),
                   pl.BlockSpec((64, 100), lambda i: (0, 0))],
        out_shape=[jax.ShapeDtypeStruct((NPAD, 100), F32),
                   jax.ShapeDtypeStruct((64, 100), F32)],
        scratch_shapes=[pltpu.VMEM((64, 100), F32), pltpu.VMEM((64, 8), F32)],
        compiler_params=pltpu.CompilerParams(
            dimension_semantics=("arbitrary",)),
    )(t, b3, *w)


def _sc_gather(table, idx):
    nrows, dd = table.shape
    per_w = idx.shape[0] // 32
    nit = per_w // G

    @functools.partial(
        pl.kernel,
        out_type=jax.ShapeDtypeStruct((idx.shape[0], dd), F32),
        mesh=_sc_mesh(),
        scratch_types=[pltpu.VMEM((G,), jnp.int32),
                       pltpu.VMEM((G,), jnp.int32),
                       pltpu.VMEM((G, dd), F32),
                       pltpu.VMEM((G, dd), F32),
                       pltpu.SemaphoreType.DMA,
                       pltpu.SemaphoreType.DMA,
                       pltpu.SemaphoreType.DMA,
                       pltpu.SemaphoreType.DMA],
    )
    def k(table_hbm, idx_hbm, out_hbm, i0, i1, r0, r1, sg0, sg1, sw0, sw1):
        wid = lax.axis_index("s") * 2 + lax.axis_index("c")
        base = wid * per_w
        bufs = ((i0, r0, sg0, sw0), (i1, r1, sg1, sw1))
        pltpu.sync_copy(idx_hbm.at[pl.ds(base, G)], i0)
        pltpu.async_copy(table_hbm.at[i0], r0, sg0)

        def grp(g, carry):
            for b in range(2):
                ib, rb, sgb, swb = bufs[b]
                ob, rob, sgo, swo = bufs[1 - b]
                i = 2 * g + b
                off = base + i * G

                @pl.when(i + 1 < nit)
                def _prefetch():
                    @pl.when(i >= 1)
                    def _wait_wb():
                        pltpu.make_async_copy(
                            rob, out_hbm.at[pl.ds(base, G)], swo).wait()
                    pltpu.sync_copy(idx_hbm.at[pl.ds(off + G, G)], ob)
                    pltpu.async_copy(table_hbm.at[ob], rob, sgo)

                pltpu.make_async_copy(table_hbm.at[ib], rb, sgb).wait()
                pltpu.async_copy(rb, out_hbm.at[pl.ds(off, G)], swb)
            return carry

        lax.fori_loop(0, nit // 2, grp, 0)
        pltpu.make_async_copy(r0, out_hbm.at[pl.ds(base, G)], sw0).wait()
        pltpu.make_async_copy(r1, out_hbm.at[pl.ds(base, G)], sw1).wait()

    return k(table, idx)


def _sc_scatter(parts, tgt, z128):
    per_w = EPAD // 32
    zrows = ZR // 16

    @functools.partial(
        pl.kernel,
        out_type=jax.ShapeDtypeStruct((2 * NR, D), F32),
        mesh=_sc_mesh(),
        scratch_types=[pltpu.VMEM((G,), jnp.int32),
                       pltpu.VMEM((G,), jnp.int32),
                       pltpu.VMEM((G, D), F32),
                       pltpu.VMEM((G, D), F32),
                       pltpu.SemaphoreType.DMA],
    )
    def k(p_hbm, tgt_hbm, z_hbm, agg_hbm, ii_v, li_v, rows_v, zbuf, sem):
        c = lax.axis_index("c")
        s = lax.axis_index("s")
        base = c * NR
        pltpu.sync_copy(z_hbm, zbuf)
        for j in range(zrows // G + 1):
            st = base + s * zrows + min(j * G, zrows - G)
            pltpu.sync_copy(zbuf, agg_hbm.at[pl.ds(st, G)])
        plsc.subcore_barrier()

        def body(i, carry):
            off = (c * 16 + s) * per_w + i * G
            pltpu.sync_copy(tgt_hbm.at[pl.ds(off, G)], ii_v)
            pltpu.sync_copy(p_hbm.at[pl.ds(off, G)], rows_v)
            for j in range(G // 16):
                li_v[pl.ds(j * 16, 16)] = ii_v[pl.ds(j * 16, 16)] + base
            pltpu.async_copy(rows_v, agg_hbm.at[li_v], sem).wait()
            return carry

        lax.fori_loop(0, per_w // G, body, 0)

    return k(parts, tgt, z128)


def _prep_weights(params):
    def row(x):
        return x.reshape(1, -1)

    ne = params['node_embed']
    w_ne = (ne['wh'], ne['ws'][0:6], ne['ws'][6:22], row(ne['bs']),
            ne['wv'], ne['wg'], row(ne['bg']))
    ee = params['edge_embed']
    w_ee = (ee['wh'], ee['ws'][0:32], row(ee['ws'][32]), row(ee['bs']),
            ee['wv'], ee['wg'].T, row(ee['bg']))
    w_msg, w_ff = [], []
    for layer in params['layers']:
        m0, m1, m2 = layer['msg']
        wm = (m0['wh'][0:16], m0['wh'][16:17], m0['wh'][17:33],
              m0['ws'][0:100], m0['ws'][100:132], m0['ws'][132:232],
              m0['ws'][232:265], row(m0['bs']), m0['wv'], m0['wg'],
              row(m0['bg']))
        for m in (m1, m2):
            wm = wm + (m['wh'], m['ws'][0:100], m['ws'][100:116],
                       row(m['bs']), m['wv'], m['wg'], row(m['bg']))
        w_msg.append(wm)
        f0, f1 = layer['ff']
        wf = (f0['wh'], f0['ws'][0:100], f0['ws'][100:132], row(f0['bs']),
              f0['wv'], f0['wg'], row(f0['bg']),
              f1['wh'], f1['ws'][0:400], f1['ws'][400:432], row(f1['bs']),
              f1['wv'], f1['wg'], row(f1['bg']))
        w_ff.append(wf)
    po = params['out']
    w_out = (po['wh'], po['ws'][0:100], po['ws'][100:116], row(po['bs']))
    return w_ne, w_ee, w_msg, w_ff, w_out


def kernel(node_s, node_v, edge_s, edge_v, edge_index, batch, params):
    i32 = jnp.int32
    w_ne, w_ee, w_msg, w_ff, w_out = _prep_weights(params)

    ns = jnp.pad(node_s, ((0, NPAD - N), (0, 2)))
    nv = jnp.pad(node_v.transpose(0, 2, 1).reshape(N, 9),
                 ((0, NPAD - N), (0, 7)))
    es = jnp.pad(edge_s, ((0, EPAD - E), (0, 0)))
    ev = jnp.pad(edge_v.reshape(E, 3), ((0, EPAD - E), (0, 5)))

    pad_ids = jnp.arange(EPAD - E, dtype=i32) % N
    srcp = jnp.concatenate([edge_index[0], pad_ids])
    dst_p = jnp.concatenate([edge_index[1],
                             jnp.full((EPAD - E,), NPAD - 1, i32)])
    dst_sorted, order = lax.sort_key_val(dst_p, jnp.arange(EPAD, dtype=i32))
    src_sorted = jnp.take(srcp, order)
    e_idx = jnp.arange(EPAD, dtype=i32)
    prev = jnp.concatenate([dst_sorted[:1] - 1, dst_sorted[:-1]])
    newseg = dst_sorted != prev
    b = (newseg & (e_idx % SB != 0)).astype(i32)
    csum = jnp.cumsum(b)
    rank = csum - jnp.take(csum, e_idx - (e_idx % SB))
    sbi = e_idx // SB
    nid = jnp.full((NSB, SB), -1, i32).at[sbi, rank].max(dst_sorted)
    continued = ~newseg[::SB]
    riota = jnp.arange(SB, dtype=i32)[None, :]
    tgt2 = jnp.where(nid < 0, NPAD + NSB + (riota % G), nid)
    tgt2 = tgt2.at[:, 0].set(jnp.where(
        continued, NPAD + jnp.arange(NSB, dtype=i32), tgt2[:, 0]))
    tgt = tgt2.reshape(EPAD)
    cid3 = dst_sorted[::SB].reshape(1, 1, NSB)
    rank3 = rank.reshape(EPAD // EB, 1, EB)
    b3 = jnp.pad(batch, (0, NPAD - N),
                 constant_values=-1).reshape(NPAD // NB, 1, NB)
    z128 = jnp.zeros((G, D), F32)

    t = _node_embed(ns, nv, w_ne)
    eemb = _sc_gather(_edge_embed(es, ev, w_ee), order)
    for l in range(3):
        g_src = _sc_gather(t, src_sorted)
        g_dst = _sc_gather(t, dst_sorted)
        m = _edge_msg(g_src, g_dst, eemb, rank3, w_msg[l])
        agg2 = _sc_scatter(m, tgt, z128)
        t = _node_update(t, agg2, cid3, w_ff[l])
    residue_pad, graph_emb = _out_pool(t, b3, w_out)
    return (graph_emb, residue_pad[:N])


# speedup vs baseline: 12.6932x; 1.0004x over previous
import functools

import jax
import jax.numpy as jnp
from jax import lax
from jax.experimental import pallas as pl
from jax.experimental.pallas import tpu as pltpu
from jax.experimental.pallas import tpu_sc as plsc

N = 50000
NPAD = 50176
E = 800000
EPAD = 802816
D = 256
EE_D = 128
NB = 512
EB = 2048
G = 128
SB = 256
NSB = EPAD // SB
NR = 75264
ZR = 53504
F32 = jnp.float32

@functools.cache
def _sc_mesh():
    return plsc.VectorSubcoreMesh(core_axis_name="c", subcore_axis_name="s")


def _ln_s(s):
    mu = jnp.mean(s, axis=1, keepdims=True)
    var = jnp.mean((s - mu) * (s - mu), axis=1, keepdims=True)
    return (s - mu) / jnp.sqrt(var + 1e-5)


def _ln_v(vp, nv):
    msq = (vp[0] * vp[0] + vp[1] * vp[1] + vp[2] * vp[2])
    vn = jnp.sqrt(jnp.sum(msq, axis=1, keepdims=True) / nv + 1e-8)
    return [p / vn for p in vp]


def _dot(a, b):
    return jnp.dot(a, b, preferred_element_type=F32)


def _gvp_core(s_lin, vh, ws_v, bs, wv, wg, bg, relu):
    vn = jnp.sqrt(vh[0] * vh[0] + vh[1] * vh[1] + vh[2] * vh[2] + 1e-8)
    so = s_lin + _dot(vn, ws_v) + bs
    if wv is None:
        return (jnp.maximum(so, 0.0) if relu else so), None
    gate = jax.nn.sigmoid(_dot(so, wg) + bg)
    vout = [_dot(h, wv) * gate for h in vh]
    so = jnp.maximum(so, 0.0) if relu else so
    return so, vout


def _pack_node_row(s, vp, ones_col):
    b = s.shape[0]
    mid = jnp.full((b, 1), 1.0, F32) if ones_col else jnp.zeros((b, 1), F32)
    return jnp.concatenate(
        [s, mid, jnp.zeros((b, 11), F32), vp[0], vp[1], vp[2],
         jnp.zeros((b, D - 160), F32)], axis=1)


def _split_node_row(t):
    s = t[:, 0:100]
    vp = [t[:, 112 + 16 * c:128 + 16 * c] for c in range(3)]
    return s, vp


def _full(shape):
    return pl.BlockSpec(shape, lambda i: (0,) * len(shape))


def _node_embed_body(s_ref, v_ref, wh, ws_s, ws_v, bs, wv, wg, bg, out_ref):
    s = s_ref[:, 0:6]
    vp = [v_ref[:, 3 * c:3 * c + 3] for c in range(3)]
    s = _ln_s(s)
    vp = _ln_v(vp, 3)
    vh = [_dot(p, wh[...]) for p in vp]
    so, vo = _gvp_core(_dot(s, ws_s[...]), vh, ws_v[...], bs[...],
                       wv[...], wg[...], bg[...], relu=False)
    out_ref[...] = _pack_node_row(so, vo, ones_col=False)


def _edge_embed_body(es_ref, ev_ref, wh, ws_s, ws_v, bs, wv, wg_t, bg, out_ref):
    es = es_ref[...]
    evp = [ev_ref[:, c:c + 1] for c in range(3)]
    es = _ln_s(es)
    evp = _ln_v(evp, 1)
    whs = wh[0, 0]
    vh = [p * whs for p in evp]
    vn = jnp.sqrt(vh[0] * vh[0] + vh[1] * vh[1] + vh[2] * vh[2] + 1e-8)
    so = _dot(es, ws_s[...]) + vn * ws_v[...] + bs[...]
    gate = jax.nn.sigmoid(
        jnp.sum(so * wg_t[...], axis=1, keepdims=True) + bg[0, 0])
    vo = [p * wv[0, 0] * gate for p in vh]
    b = so.shape[0]
    out_ref[...] = jnp.concatenate(
        [so, vo[0], vo[1], vo[2], jnp.zeros((b, EE_D - 35), F32)], axis=1)


def _edge_msg_body(gs_ref, gd_ref, ee_ref, rank_ref,
                   m0_wh_s, m0_wh_e, m0_wh_d, m0_ws_ss, m0_ws_es, m0_ws_ds,
                   m0_ws_v, m0_bs, m0_wv, m0_wg, m0_bg,
                   m1_wh, m1_ws_s, m1_ws_v, m1_bs, m1_wv, m1_wg, m1_bg,
                   m2_wh, m2_ws_s, m2_ws_v, m2_bs, m2_wv, m2_wg, m2_bg,
                   out_ref):
    ssrc, vs = _split_node_row(gs_ref[...])
    sdst, vd = _split_node_row(gd_ref[...])
    ee = ee_ref[...]
    es = ee[:, 0:32]
    evc = [ee[:, 32 + c:33 + c] for c in range(3)]
    vh = [_dot(vs[c], m0_wh_s[...]) + evc[c] * m0_wh_e[...]
          + _dot(vd[c], m0_wh_d[...]) for c in range(3)]
    s_lin = _dot(ssrc, m0_ws_ss[...]) + _dot(es, m0_ws_es[...]) \
        + _dot(sdst, m0_ws_ds[...])
    s, v = _gvp_core(s_lin, vh, m0_ws_v[...], m0_bs[...],
                     m0_wv[...], m0_wg[...], m0_bg[...], relu=True)
    vh = [_dot(v[c], m1_wh[...]) for c in range(3)]
    s, v = _gvp_core(_dot(s, m1_ws_s[...]), vh, m1_ws_v[...], m1_bs[...],
                     m1_wv[...], m1_wg[...], m1_bg[...], relu=True)
    vh = [_dot(v[c], m2_wh[...]) for c in range(3)]
    s, v = _gvp_core(_dot(s, m2_ws_s[...]), vh, m2_ws_v[...], m2_bs[...],
                     m2_wv[...], m2_wg[...], m2_bg[...], relu=False)
    msgs = _pack_node_row(s, v, ones_col=True)
    rk = rank_ref[0, 0, :]
    riota = lax.broadcasted_iota(jnp.int32, (SB, 1), 0)
    parts = []
    for s8 in range(EB // SB):
        oh = (riota == rk[s8 * SB:(s8 + 1) * SB].reshape(1, SB)).astype(F32)
        parts.append(_dot(oh, msgs[s8 * SB:(s8 + 1) * SB, :]))
    out_ref[...] = jnp.concatenate(parts, axis=0)


def _node_update_body(t_ref, a0_ref, a1_ref, c0_ref, c1_ref, cid_ref,
                      f0_wh, f0_ws_s, f0_ws_v, f0_bs, f0_wv, f0_wg, f0_bg,
                      f1_wh, f1_ws_s, f1_ws_v, f1_bs, f1_wv, f1_wg, f1_bg,
                      out_ref):
    s0, vp0 = _split_node_row(t_ref[...])
    car = c0_ref[...] + c1_ref[...]
    ids = cid_ref[0]
    base = pl.program_id(0) * NB
    oh = (lax.broadcasted_iota(jnp.int32, (NB, 1), 0) + base == ids)
    agg = a0_ref[...] + a1_ref[...] + _dot(oh.astype(F32), car)
    deg = jnp.maximum(agg[:, 100:101], 1.0)
    s = s0 + agg[:, 0:100] / deg
    vp = [vp0[c] + agg[:, 112 + 16 * c:128 + 16 * c] / deg for c in range(3)]
    s = _ln_s(s)
    vp = _ln_v(vp, 16)
    vh = [_dot(p, f0_wh[...]) for p in vp]
    fs, fv = _gvp_core(_dot(s, f0_ws_s[...]), vh, f0_ws_v[...], f0_bs[...],
                       f0_wv[...], f0_wg[...], f0_bg[...], relu=True)
    vh = [_dot(fv[c], f1_wh[...]) for c in range(3)]
    fs, fv = _gvp_core(_dot(fs, f1_ws_s[...]), vh, f1_ws_v[...], f1_bs[...],
                       f1_wv[...], f1_wg[...], f1_bg[...], relu=False)
    s = _ln_s(s + fs)
    vp = _ln_v([vp[c] + fv[c] for c in range(3)], 16)
    out_ref[...] = _pack_node_row(s, vp, ones_col=False)


def _out_pool_body(t_ref, b_ref, wh, ws_s, ws_v, bs,
                   res_ref, emb_ref, acc, cnt):
    i = pl.program_id(0)

    @pl.when(i == 0)
    def _init():
        acc[...] = jnp.zeros_like(acc)
        cnt[...] = jnp.zeros_like(cnt)

    s, vp = _split_node_row(t_ref[...])
    s = _ln_s(s)
    vp = _ln_v(vp, 16)
    vh = [_dot(p, wh[...]) for p in vp]
    res, _ = _gvp_core(_dot(s, ws_s[...]), vh, ws_v[...], bs[...],
                       None, None, None, relu=True)
    res_ref[...] = res
    bid = b_ref[0]
    gid = lax.broadcasted_iota(jnp.int32, (64, 1), 0)
    onehot_t = (gid == bid).astype(F32)
    acc[...] = acc[...] + _dot(onehot_t, res)
    cnt[...] = cnt[...] + _dot(onehot_t, jnp.ones((res.shape[0], 8), F32))
    emb_ref[...] = acc[...] / jnp.maximum(cnt[:, 0:1], 1.0)


def _node_embed(ns, nv, w):
    grid = NPAD // NB
    return pl.pallas_call(
        _node_embed_body,
        grid=(grid,),
        in_specs=[pl.BlockSpec((NB, 8), lambda i: (i, 0)),
                  pl.BlockSpec((NB, 16), lambda i: (i, 0))]
        + [_full(x.shape) for x in w],
        out_specs=pl.BlockSpec((NB, D), lambda i: (i, 0)),
        out_shape=jax.ShapeDtypeStruct((NPAD, D), F32),
    )(ns, nv, *w)


def _edge_embed(es, ev, w):
    grid = EPAD // EB
    return pl.pallas_call(
        _edge_embed_body,
        grid=(grid,),
        in_specs=[pl.BlockSpec((EB, 32), lambda i: (i, 0)),
                  pl.BlockSpec((EB, 8), lambda i: (i, 0))]
        + [_full(x.shape) for x in w],
        out_specs=pl.BlockSpec((EB, EE_D), lambda i: (i, 0)),
        out_shape=jax.ShapeDtypeStruct((EPAD, EE_D), F32),
    )(es, ev, *w)


def _edge_msg(gs, gd, ee, rank3, w):
    grid = EPAD // EB
    return pl.pallas_call(
        _edge_msg_body,
        grid=(grid,),
        in_specs=[pl.BlockSpec((EB, D), lambda i: (i, 0)),
                  pl.BlockSpec((EB, D), lambda i: (i, 0)),
                  pl.BlockSpec((EB, EE_D), lambda i: (i, 0)),
                  pl.BlockSpec((1, 1, EB), lambda i: (i, 0, 0))]
        + [_full(x.shape) for x in w],
        out_specs=pl.BlockSpec((EB, D), lambda i: (i, 0)),
        out_shape=jax.ShapeDtypeStruct((EPAD, D), F32),
    )(gs, gd, ee, rank3, *w)


def _node_update(t, agg2, cid3, w):
    grid = NPAD // NB
    return pl.pallas_call(
        _node_update_body,
        grid=(grid,),
        in_specs=[pl.BlockSpec((NB, D), lambda i: (i, 0)),
                  pl.BlockSpec((NB, D), lambda i: (i, 0)),
                  pl.BlockSpec((NB, D), lambda i: (i + NR // NB, 0)),
                  pl.BlockSpec((NSB, D), lambda i: (NPAD // NSB, 0)),
                  pl.BlockSpec((NSB, D), lambda i: ((NR + NPAD) // NSB, 0)),
                  pl.BlockSpec((1, 1, NSB), lambda i: (0, 0, 0))]
        + [_full(x.shape) for x in w],
        out_specs=pl.BlockSpec((NB, D), lambda i: (i, 0)),
        out_shape=jax.ShapeDtypeStruct((NPAD, D), F32),
    )(t, agg2, agg2, agg2, agg2, cid3, *w)


def _out_pool(t, b3, w):
    grid = NPAD // NB
    return pl.pallas_call(
        _out_pool_body,
        grid=(grid,),
        in_specs=[pl.BlockSpec((NB, D), lambda i: (i, 0)),
                  pl.BlockSpec((1, 1, NB), lambda i: (i, 0, 0))]
        + [_full(x.shape) for x in w],
        out_specs=[pl.BlockSpec((NB, 100), lambda i: (i, 0)),
                   pl.BlockSpec((64, 100), lambda i: (0, 0))],
        out_shape=[jax.ShapeDtypeStruct((NPAD, 100), F32),
                   jax.ShapeDtypeStruct((64, 100), F32)],
        scratch_shapes=[pltpu.VMEM((64, 100), F32), pltpu.VMEM((64, 8), F32)],
        compiler_params=pltpu.CompilerParams(
            dimension_semantics=("arbitrary",)),
    )(t, b3, *w)


def _sc_gather(table, idx):
    nrows, dd = table.shape
    per_w = idx.shape[0] // 32
    nit = per_w // G

    @functools.partial(
        pl.kernel,
        out_type=jax.ShapeDtypeStruct((idx.shape[0], dd), F32),
        mesh=_sc_mesh(),
        scratch_types=[pltpu.VMEM((G,), jnp.int32),
                       pltpu.VMEM((G,), jnp.int32),
                       pltpu.VMEM((G, dd), F32),
                       pltpu.VMEM((G, dd), F32),
                       pltpu.SemaphoreType.DMA,
                       pltpu.SemaphoreType.DMA,
                       pltpu.SemaphoreType.DMA,
                       pltpu.SemaphoreType.DMA],
        compiler_params=pltpu.CompilerParams(use_tc_tiling_on_sc=True),
    )
    def k(table_hbm, idx_hbm, out_hbm, i0, i1, r0, r1, sg0, sg1, sw0, sw1):
        wid = lax.axis_index("s") * 2 + lax.axis_index("c")
        base = wid * per_w
        bufs = ((i0, r0, sg0, sw0), (i1, r1, sg1, sw1))
        pltpu.sync_copy(idx_hbm.at[pl.ds(base, G)], i0)
        pltpu.async_copy(table_hbm.at[i0], r0, sg0)

        def grp(g, carry):
            for b in range(2):
                ib, rb, sgb, swb = bufs[b]
                ob, rob, sgo, swo = bufs[1 - b]
                i = 2 * g + b
                off = base + i * G

                @pl.when(i + 1 < nit)
                def _prefetch():
                    @pl.when(i >= 1)
                    def _wait_wb():
                        pltpu.make_async_copy(
                            rob, out_hbm.at[pl.ds(base, G)], swo).wait()
                    pltpu.sync_copy(idx_hbm.at[pl.ds(off + G, G)], ob)
                    pltpu.async_copy(table_hbm.at[ob], rob, sgo)

                pltpu.make_async_copy(table_hbm.at[ib], rb, sgb).wait()
                pltpu.async_copy(rb, out_hbm.at[pl.ds(off, G)], swb)
            return carry

        lax.fori_loop(0, nit // 2, grp, 0)
        pltpu.make_async_copy(r0, out_hbm.at[pl.ds(base, G)], sw0).wait()
        pltpu.make_async_copy(r1, out_hbm.at[pl.ds(base, G)], sw1).wait()

    return k(table, idx)


def _sc_scatter(parts, tgt, z128):
    per_w = EPAD // 32
    zrows = ZR // 16

    @functools.partial(
        pl.kernel,
        out_type=jax.ShapeDtypeStruct((2 * NR, D), F32),
        mesh=_sc_mesh(),
        scratch_types=[pltpu.VMEM((G,), jnp.int32),
                       pltpu.VMEM((G,), jnp.int32),
                       pltpu.VMEM((G, D), F32),
                       pltpu.VMEM((G, D), F32),
                       pltpu.SemaphoreType.DMA],
        compiler_params=pltpu.CompilerParams(use_tc_tiling_on_sc=True),
    )
    def k(p_hbm, tgt_hbm, z_hbm, agg_hbm, ii_v, li_v, rows_v, zbuf, sem):
        c = lax.axis_index("c")
        s = lax.axis_index("s")
        base = c * NR
        pltpu.sync_copy(z_hbm, zbuf)
        for j in range(zrows // G + 1):
            st = base + s * zrows + min(j * G, zrows - G)
            pltpu.sync_copy(zbuf, agg_hbm.at[pl.ds(st, G)])
        plsc.subcore_barrier()

        def body(i, carry):
            off = (c * 16 + s) * per_w + i * G
            pltpu.sync_copy(tgt_hbm.at[pl.ds(off, G)], ii_v)
            pltpu.sync_copy(p_hbm.at[pl.ds(off, G)], rows_v)
            for j in range(G // 16):
                li_v[pl.ds(j * 16, 16)] = ii_v[pl.ds(j * 16, 16)] + base
            pltpu.async_copy(rows_v, agg_hbm.at[li_v], sem).wait()
            return carry

        lax.fori_loop(0, per_w // G, body, 0)

    return k(parts, tgt, z128)


def _prep_weights(params):
    def row(x):
        return x.reshape(1, -1)

    ne = params['node_embed']
    w_ne = (ne['wh'], ne['ws'][0:6], ne['ws'][6:22], row(ne['bs']),
            ne['wv'], ne['wg'], row(ne['bg']))
    ee = params['edge_embed']
    w_ee = (ee['wh'], ee['ws'][0:32], row(ee['ws'][32]), row(ee['bs']),
            ee['wv'], ee['wg'].T, row(ee['bg']))
    w_msg, w_ff = [], []
    for layer in params['layers']:
        m0, m1, m2 = layer['msg']
        wm = (m0['wh'][0:16], m0['wh'][16:17], m0['wh'][17:33],
              m0['ws'][0:100], m0['ws'][100:132], m0['ws'][132:232],
              m0['ws'][232:265], row(m0['bs']), m0['wv'], m0['wg'],
              row(m0['bg']))
        for m in (m1, m2):
            wm = wm + (m['wh'], m['ws'][0:100], m['ws'][100:116],
                       row(m['bs']), m['wv'], m['wg'], row(m['bg']))
        w_msg.append(wm)
        f0, f1 = layer['ff']
        wf = (f0['wh'], f0['ws'][0:100], f0['ws'][100:132], row(f0['bs']),
              f0['wv'], f0['wg'], row(f0['bg']),
              f1['wh'], f1['ws'][0:400], f1['ws'][400:432], row(f1['bs']),
              f1['wv'], f1['wg'], row(f1['bg']))
        w_ff.append(wf)
    po = params['out']
    w_out = (po['wh'], po['ws'][0:100], po['ws'][100:116], row(po['bs']))
    return w_ne, w_ee, w_msg, w_ff, w_out


def kernel(node_s, node_v, edge_s, edge_v, edge_index, batch, params):
    i32 = jnp.int32
    w_ne, w_ee, w_msg, w_ff, w_out = _prep_weights(params)

    ns = jnp.pad(node_s, ((0, NPAD - N), (0, 2)))
    nv = jnp.pad(node_v.transpose(0, 2, 1).reshape(N, 9),
                 ((0, NPAD - N), (0, 7)))
    es = jnp.pad(edge_s, ((0, EPAD - E), (0, 0)))
    ev = jnp.pad(edge_v.reshape(E, 3), ((0, EPAD - E), (0, 5)))

    pad_ids = jnp.arange(EPAD - E, dtype=i32) % N
    srcp = jnp.concatenate([edge_index[0], pad_ids])
    dst_p = jnp.concatenate([edge_index[1],
                             jnp.full((EPAD - E,), NPAD - 1, i32)])
    dst_sorted, order = lax.sort_key_val(dst_p, jnp.arange(EPAD, dtype=i32))
    src_sorted = jnp.take(srcp, order)
    e_idx = jnp.arange(EPAD, dtype=i32)
    prev = jnp.concatenate([dst_sorted[:1] - 1, dst_sorted[:-1]])
    newseg = dst_sorted != prev
    b = (newseg & (e_idx % SB != 0)).astype(i32)
    csum = jnp.cumsum(b)
    rank = csum - jnp.take(csum, e_idx - (e_idx % SB))
    sbi = e_idx // SB
    nid = jnp.full((NSB, SB), -1, i32).at[sbi, rank].max(dst_sorted)
    continued = ~newseg[::SB]
    riota = jnp.arange(SB, dtype=i32)[None, :]
    tgt2 = jnp.where(nid < 0, NPAD + NSB + (riota % G), nid)
    tgt2 = tgt2.at[:, 0].set(jnp.where(
        continued, NPAD + jnp.arange(NSB, dtype=i32), tgt2[:, 0]))
    tgt = tgt2.reshape(EPAD)
    cid3 = dst_sorted[::SB].reshape(1, 1, NSB)
    rank3 = rank.reshape(EPAD // EB, 1, EB)
    b3 = jnp.pad(batch, (0, NPAD - N),
                 constant_values=-1).reshape(NPAD // NB, 1, NB)
    z128 = jnp.zeros((G, D), F32)

    t = _node_embed(ns, nv, w_ne)
    eemb = _sc_gather(_edge_embed(es, ev, w_ee), order)
    for l in range(3):
        g_src = _sc_gather(t, src_sorted)
        g_dst = _sc_gather(t, dst_sorted)
        m = _edge_msg(g_src, g_dst, eemb, rank3, w_msg[l])
        agg2 = _sc_scatter(m, tgt, z128)
        t = _node_update(t, agg2, cid3, w_ff[l])
    residue_pad, graph_emb = _out_pool(t, b3, w_out)
    return (graph_emb, residue_pad[:N])
